# baseline (speedup 1.0000x reference)
import jax
import jax.numpy as jnp
from jax import lax
from jax.experimental import pallas as pl
from jax.experimental.pallas import tpu as pltpu
from jax.experimental.pallas import tpu_sc as plsc

_NC = 2
_NS = 16
_L = 16

_N_WEIGHTS = 100000
_TAB = 100096
_B, _T, _F = 4096, 200, 8
_BT = _B // 128
_TI = _T // 8
_CH_TI = 5
_N_CHUNKS = _TI // _CH_TI
_CH_T = _CH_TI * 8
_CH_GROUPS = _CH_T * 128 // _L
_PER_SUB = _TAB // _NS
_PGROUPS = _PER_SUB // _L


def _body(xv_hbm, lw_hbm, out_hbm, wtab, xbufs, obufs, pbuf, pall, hstats,
          sem_t, sems_x, sems_o):
    c = lax.axis_index("c")
    s = lax.axis_index("s")
    w = c * _NS + s

    cp_t = pltpu.async_copy(lw_hbm, wtab, sem_t)
    cp_x = [None, None]
    cp_x[0] = pltpu.async_copy(
        xv_hbm.at[pl.ds(0, _CH_T), w, 0, :], xbufs[0], sems_x[0]
    )
    ninf = jnp.full((_L,), -jnp.inf, jnp.float32)
    cp_t.wait()

    base_p = s * _PER_SUB

    def max_body(k, mv):
        return jnp.maximum(mv, wtab[pl.ds(base_p + k * _L, _L)])

    m_vec = lax.fori_loop(0, _PGROUPS, max_body, ninf)
    m_loc = jnp.max(m_vec)

    def sum_body(k, acc):
        return acc + jnp.exp(wtab[pl.ds(base_p + k * _L, _L)] - m_loc)

    s_vec = lax.fori_loop(0, _PGROUPS, sum_body, jnp.zeros((_L,), jnp.float32))
    s_loc = jnp.sum(s_vec)

    lane = lax.iota(jnp.int32, _L)
    pbuf[...] = jnp.where(lane == 0, m_loc, jnp.where(lane == 1, s_loc, 0.0))
    pltpu.async_copy(pbuf, hstats.at[c, s], sem_t).wait()
    plsc.subcore_barrier()
    pltpu.async_copy(hstats.at[c], pall, sem_t).wait()
    m_all = plsc.load_gather(pall, [lane, jnp.zeros((_L,), jnp.int32)])
    s_all = plsc.load_gather(pall, [lane, jnp.ones((_L,), jnp.int32)])
    gmax = jnp.max(m_all)
    ssum = jnp.zeros((_L,), jnp.float32) + jnp.sum(
        s_all * jnp.exp(m_all - gmax)
    )
    seed = lax.bitcast_convert_type(
        jnp.int32(0x7EF311C3) - lax.bitcast_convert_type(ssum, jnp.int32),
        jnp.float32,
    )
    rsum = seed
    for _ in range(3):
        rsum = rsum * (2.0 - ssum * rsum)

    cp_o = [None, None]
    for i in range(_N_CHUNKS):
        cur = i % 2
        nxt = 1 - cur
        if i + 1 < _N_CHUNKS:
            cp_x[nxt] = pltpu.async_copy(
                xv_hbm.at[pl.ds((i + 1) * _CH_T, _CH_T), w, 0, :],
                xbufs[nxt],
                sems_x[nxt],
            )
        cp_x[cur].wait()
        if cp_o[cur] is not None:
            cp_o[cur].wait()
        xbuf = xbufs[cur]
        obuf = obufs[cur]

        def grp(j, _):
            row = j // 8
            col = (j % 8) * _L
            idxv = xbuf[row, pl.ds(col, _L)].astype(jnp.int32)
            g = plsc.load_gather(wtab, [idxv])
            obuf[j // 64, (j // 8) % 8, pl.ds(col, _L)] = (
                jnp.exp(g - gmax) * rsum
            )
            return 0

        lax.fori_loop(0, _CH_GROUPS, grp, 0, unroll=8)
        cp_o[cur] = pltpu.async_copy(
            obuf, out_hbm.at[pl.ds(i * _CH_TI, _CH_TI), w, :, :], sems_o[cur]
        )
    for cp in cp_o:
        if cp is not None:
            cp.wait()


@jax.jit
def kernel(x, logit_weights):
    xv = x.transpose(1, 0, 2).reshape(_T, _BT, 128, _F).transpose(0, 1, 3, 2)

    mesh = plsc.VectorSubcoreMesh(core_axis_name="c", subcore_axis_name="s")
    sc = pl.kernel(
        _body,
        out_type=jax.ShapeDtypeStruct((_TI, _BT, 8, 128), jnp.float32),
        mesh=mesh,
        scratch_types=[
            pltpu.VMEM((_TAB,), jnp.float32),
            [pltpu.VMEM((_CH_T, 128), jnp.float32) for _ in range(2)],
            [pltpu.VMEM((_CH_TI, 8, 128), jnp.float32) for _ in range(2)],
            pltpu.VMEM((_L,), jnp.float32),
            pltpu.VMEM((_NS, _L), jnp.float32),
            pltpu.HBM((_NC, _NS, _L), jnp.float32),
            pltpu.SemaphoreType.DMA,
            [pltpu.SemaphoreType.DMA for _ in range(2)],
            [pltpu.SemaphoreType.DMA for _ in range(2)],
        ],
        compiler_params=pltpu.CompilerParams(needs_layout_passes=False),
    )
    lw_pad = jnp.pad(logit_weights, (0, _TAB - _N_WEIGHTS),
                     constant_values=-1e30)
    res = sc(xv, lw_pad)
    return res.transpose(1, 3, 0, 2).reshape(_B, _T)


# speedup vs baseline: 1.0171x; 1.0171x over previous
import jax
import jax.numpy as jnp
from jax import lax
from jax.experimental import pallas as pl
from jax.experimental.pallas import tpu as pltpu
from jax.experimental.pallas import tpu_sc as plsc

_NC = 2
_NS = 16
_L = 16

_N_WEIGHTS = 100000
_TAB = 100352
_B, _T, _F = 4096, 200, 8
_BT = _B // 128
_TI = _T // 8
_CH_TI = 5
_N_CHUNKS = _TI // _CH_TI
_CH_T = _CH_TI * 8
_CH_GROUPS = _CH_T * 128 // _L
_PER_SUB = _TAB // _NS
_PGROUPS = _PER_SUB // _L
_PACC = 8
_PITER = _PGROUPS // _PACC


def _body(xv_hbm, lw_hbm, out_hbm, wtab, xbufs, obufs, pbuf, pall, hstats,
          sem_t, sems_x, sems_o):
    c = lax.axis_index("c")
    s = lax.axis_index("s")
    w = c * _NS + s

    cp_t = pltpu.async_copy(lw_hbm, wtab, sem_t)
    cp_x = [None, None]
    cp_x[0] = pltpu.async_copy(
        xv_hbm.at[pl.ds(0, _CH_T), w, 0, :], xbufs[0], sems_x[0]
    )
    ninf = jnp.full((_L,), -jnp.inf, jnp.float32)
    cp_t.wait()

    base_p = s * _PER_SUB

    def max_body(k, mvs):
        return tuple(
            jnp.maximum(mvs[a], wtab[pl.ds(base_p + (k * _PACC + a) * _L, _L)])
            for a in range(_PACC)
        )

    mvs = lax.fori_loop(0, _PITER, max_body, (ninf,) * _PACC)
    m_vec = mvs[0]
    for a in range(1, _PACC):
        m_vec = jnp.maximum(m_vec, mvs[a])
    m_loc = jnp.max(m_vec)

    def sum_body(k, accs):
        return tuple(
            accs[a]
            + jnp.exp(wtab[pl.ds(base_p + (k * _PACC + a) * _L, _L)] - m_loc)
            for a in range(_PACC)
        )

    accs = lax.fori_loop(
        0, _PITER, sum_body, (jnp.zeros((_L,), jnp.float32),) * _PACC
    )
    s_vec = accs[0]
    for a in range(1, _PACC):
        s_vec = s_vec + accs[a]
    s_loc = jnp.sum(s_vec)

    lane = lax.iota(jnp.int32, _L)
    pbuf[...] = jnp.where(lane == 0, m_loc, jnp.where(lane == 1, s_loc, 0.0))
    pltpu.async_copy(pbuf, hstats.at[c, s], sem_t).wait()
    plsc.subcore_barrier()
    pltpu.async_copy(hstats.at[c], pall, sem_t).wait()
    m_all = plsc.load_gather(pall, [lane, jnp.zeros((_L,), jnp.int32)])
    s_all = plsc.load_gather(pall, [lane, jnp.ones((_L,), jnp.int32)])
    gmax = jnp.max(m_all)
    ssum = jnp.zeros((_L,), jnp.float32) + jnp.sum(
        s_all * jnp.exp(m_all - gmax)
    )
    seed = lax.bitcast_convert_type(
        jnp.int32(0x7EF311C3) - lax.bitcast_convert_type(ssum, jnp.int32),
        jnp.float32,
    )
    rsum = seed
    for _ in range(3):
        rsum = rsum * (2.0 - ssum * rsum)

    cp_o = [None, None]
    for i in range(_N_CHUNKS):
        cur = i % 2
        nxt = 1 - cur
        if i + 1 < _N_CHUNKS:
            cp_x[nxt] = pltpu.async_copy(
                xv_hbm.at[pl.ds((i + 1) * _CH_T, _CH_T), w, 0, :],
                xbufs[nxt],
                sems_x[nxt],
            )
        cp_x[cur].wait()
        if cp_o[cur] is not None:
            cp_o[cur].wait()
        xbuf = xbufs[cur]
        obuf = obufs[cur]

        def grp(j, _):
            row = j // 8
            col = (j % 8) * _L
            idxv = xbuf[row, pl.ds(col, _L)].astype(jnp.int32)
            g = plsc.load_gather(wtab, [idxv])
            obuf[j // 64, (j // 8) % 8, pl.ds(col, _L)] = (
                jnp.exp(g - gmax) * rsum
            )
            return 0

        lax.fori_loop(0, _CH_GROUPS, grp, 0, unroll=16)
        cp_o[cur] = pltpu.async_copy(
            obuf, out_hbm.at[pl.ds(i * _CH_TI, _CH_TI), w, :, :], sems_o[cur]
        )
    for cp in cp_o:
        if cp is not None:
            cp.wait()


@jax.jit
def kernel(x, logit_weights):
    xv = x.transpose(1, 0, 2).reshape(_T, _BT, 128, _F).transpose(0, 1, 3, 2)

    mesh = plsc.VectorSubcoreMesh(core_axis_name="c", subcore_axis_name="s")
    sc = pl.kernel(
        _body,
        out_type=jax.ShapeDtypeStruct((_TI, _BT, 8, 128), jnp.float32),
        mesh=mesh,
        scratch_types=[
            pltpu.VMEM((_TAB,), jnp.float32),
            [pltpu.VMEM((_CH_T, 128), jnp.float32) for _ in range(2)],
            [pltpu.VMEM((_CH_TI, 8, 128), jnp.float32) for _ in range(2)],
            pltpu.VMEM((_L,), jnp.float32),
            pltpu.VMEM((_NS, _L), jnp.float32),
            pltpu.HBM((_NC, _NS, _L), jnp.float32),
            pltpu.SemaphoreType.DMA,
            [pltpu.SemaphoreType.DMA for _ in range(2)],
            [pltpu.SemaphoreType.DMA for _ in range(2)],
        ],
        compiler_params=pltpu.CompilerParams(needs_layout_passes=False),
    )
    lw_pad = jnp.pad(logit_weights, (0, _TAB - _N_WEIGHTS),
                     constant_values=-1e30)
    res = sc(xv, lw_pad)
    return res.transpose(1, 3, 0, 2).reshape(_B, _T)


# speedup vs baseline: 1.5208x; 1.4952x over previous
import jax
import jax.numpy as jnp
from jax import lax
from jax.experimental import pallas as pl
from jax.experimental.pallas import tpu as pltpu
from jax.experimental.pallas import tpu_sc as plsc

_NC = 2
_NS = 16
_L = 16

_N_WEIGHTS = 100000
_PAD_W = 100352
_B, _T, _F = 4096, 200, 8
_BT = _B // 128
_TI = _T // 8
_CH_TI = 5
_N_CHUNKS = _TI // _CH_TI
_CH_T = _CH_TI * 8
_CH_GROUPS = _CH_T * 128 // _L


def _softmax_body(lw_ref, out_ref):
    v = lw_ref[...]
    m = jnp.max(v)
    e = jnp.exp(v - m)
    out_ref[...] = e * (1.0 / jnp.sum(e))


def _softmax_tc(lw_pad):
    return pl.pallas_call(
        _softmax_body,
        out_shape=jax.ShapeDtypeStruct((_PAD_W // 128, 128), jnp.float32),
    )(lw_pad)


def _body(xv_hbm, w_hbm, out_hbm, wtab, stab, xbufs, obufs, sem_t, sems_x,
          sems_o):
    c = lax.axis_index("c")
    s = lax.axis_index("s")
    w = c * _NS + s

    cp_x = [None, None]
    cp_x[0] = pltpu.async_copy(
        xv_hbm.at[pl.ds(0, _CH_T), w, 0, :], xbufs[0], sems_x[0]
    )

    @pl.when(s == 0)
    def _stage():
        pltpu.async_copy(w_hbm, stab, sem_t).wait()

    plsc.subcore_barrier()
    pltpu.async_copy(stab, wtab, sem_t).wait()

    cp_o = [None, None]
    for i in range(_N_CHUNKS):
        cur = i % 2
        nxt = 1 - cur
        if i + 1 < _N_CHUNKS:
            cp_x[nxt] = pltpu.async_copy(
                xv_hbm.at[pl.ds((i + 1) * _CH_T, _CH_T), w, 0, :],
                xbufs[nxt],
                sems_x[nxt],
            )
        cp_x[cur].wait()
        if cp_o[cur] is not None:
            cp_o[cur].wait()
        xbuf = xbufs[cur]
        obuf = obufs[cur]

        def grp(j, _):
            row = j // 8
            col = (j % 8) * _L
            idxv = xbuf[row, pl.ds(col, _L)].astype(jnp.int32)
            g = plsc.load_gather(wtab, [idxv])
            obuf[j // 64, (j // 8) % 8, pl.ds(col, _L)] = g
            return 0

        lax.fori_loop(0, _CH_GROUPS, grp, 0, unroll=8)
        cp_o[cur] = pltpu.async_copy(
            obuf, out_hbm.at[pl.ds(i * _CH_TI, _CH_TI), w, :, :], sems_o[cur]
        )
    for cp in cp_o:
        if cp is not None:
            cp.wait()


@jax.jit
def kernel(x, logit_weights):
    lw_pad = jnp.pad(
        logit_weights, (0, _PAD_W - _N_WEIGHTS), constant_values=-jnp.inf
    ).reshape(_PAD_W // 128, 128)
    wts = _softmax_tc(lw_pad).reshape(_PAD_W)

    xv = x.transpose(1, 0, 2).reshape(_T, _BT, 128, _F).transpose(0, 1, 3, 2)

    mesh = plsc.VectorSubcoreMesh(core_axis_name="c", subcore_axis_name="s")
    sc = pl.kernel(
        _body,
        out_type=jax.ShapeDtypeStruct((_TI, _BT, 8, 128), jnp.float32),
        mesh=mesh,
        scratch_types=[
            pltpu.VMEM((_PAD_W,), jnp.float32),
            pltpu.VMEM_SHARED((_PAD_W,), jnp.float32),
            [pltpu.VMEM((_CH_T, 128), jnp.float32) for _ in range(2)],
            [pltpu.VMEM((_CH_TI, 8, 128), jnp.float32) for _ in range(2)],
            pltpu.SemaphoreType.DMA,
            [pltpu.SemaphoreType.DMA for _ in range(2)],
            [pltpu.SemaphoreType.DMA for _ in range(2)],
        ],
        compiler_params=pltpu.CompilerParams(needs_layout_passes=False),
    )
    res = sc(xv, wts)
    return res.transpose(1, 3, 0, 2).reshape(_B, _T)
